# baseline (speedup 1.0000x reference)
import jax
import jax.numpy as jnp
from jax import lax
from jax.experimental import pallas as pl
from jax.experimental.pallas import tpu as pltpu

ALPHA_C = 0.1
K_NN = 4
RADIUS_C = 0.07
H2 = 0.03 ** 2
EPS_C = 1e-12

B, C, N = 16, 3, 2048
C8 = 8
ROWS = 256
NBLK = N // ROWS


def _loss_kernel(gt_row_ref, pred_row_ref, pt_ref,
                 gtb_row_ref, predb_row_ref, predb_ref,
                 rinv_ref, out_ref, colmin_ref):
    b = pl.program_id(0)
    i = pl.program_id(1)

    @pl.when((b == 0) & (i == 0))
    def _init_out():
        out_ref[...] = jnp.zeros((1, 1), jnp.float32)

    @pl.when(i == 0)
    def _init_colmin():
        colmin_ref[...] = jnp.full((1, N), jnp.inf, jnp.float32)

    g = gt_row_ref[0]
    q = pred_row_ref[0]
    pt = pt_ref[0]
    gb = gtb_row_ref[0]
    qb = predb_row_ref[0]
    pb = predb_ref[0]

    g2 = 0.25 * jnp.sum(g * g, axis=1, keepdims=True)
    q2 = 0.25 * jnp.sum(q * q, axis=1, keepdims=True)
    p2 = jnp.sum(pt * pt, axis=1)[None, :]

    ab = jnp.dot(jnp.concatenate([gb, qb], axis=0), pb,
                 preferred_element_type=jnp.float32)
    d_n = (g2 + p2) + ab[:ROWS]
    dpp_n = (q2 + p2) + ab[ROWS:]

    rinv = rinv_ref[0, 0, 0]
    inv_bn = 1.0 / (B * N)

    rowmin = jnp.min(d_n, axis=1)
    colmin_ref[...] = jnp.minimum(colmin_ref[...],
                                  jnp.min(d_n, axis=0, keepdims=True))
    acc = (0.8 * inv_bn) * rinv * jnp.sum(rowmin)

    qa = -0.5 * q
    m = jnp.min(dpp_n, axis=1, keepdims=True)
    dpp_n = jnp.where(dpp_n == m, jnp.inf, dpp_n)
    rep = jnp.zeros((), jnp.float32)
    for _ in range(K_NN):
        m = jnp.min(dpp_n, axis=1, keepdims=True)
        sel = dpp_n == m
        c = lax.dot_general(sel.astype(jnp.float32), pt,
                            (((1,), (0,)), ((), ())),
                            precision=lax.Precision.HIGHEST)
        dpp_n = jnp.where(sel, jnp.inf, dpp_n)
        diff = c - qa
        d2 = jnp.maximum(jnp.sum(diff * diff, axis=1), EPS_C)
        dist = jnp.sqrt(d2)
        w = jnp.exp(-d2 * (1.0 / H2))
        rep = rep + jnp.sum((RADIUS_C - dist) * w)
    acc = acc + (ALPHA_C * inv_bn / K_NN) * rep

    tail = jnp.where(i == NBLK - 1,
                     (0.2 * inv_bn) * rinv * jnp.sum(colmin_ref[...]),
                     0.0)
    out_ref[...] = out_ref[...] + (acc + tail)


def kernel(pred, gt, pcd_radius):
    pad_t = [(0, 0), (0, 0), (0, C8 - C)]
    gt_t = jnp.pad(jnp.transpose(-2.0 * gt, (0, 2, 1)), pad_t)
    pred_t = jnp.pad(jnp.transpose(-2.0 * pred, (0, 2, 1)), pad_t)
    pt = jnp.pad(jnp.transpose(pred, (0, 2, 1)), pad_t)
    gtb_t = gt_t.astype(jnp.bfloat16)
    predb_t = pred_t.astype(jnp.bfloat16)
    predb = jnp.pad(pred, [(0, 0), (0, C8 - C), (0, 0)]).astype(jnp.bfloat16)
    rinv = (1.0 / pcd_radius).reshape(B, 1, 1)
    row_spec = pl.BlockSpec((1, ROWS, C8), lambda b, i: (b, i, 0))
    pts_spec = pl.BlockSpec((1, N, C8), lambda b, i: (b, 0, 0))
    col_spec = pl.BlockSpec((1, C8, N), lambda b, i: (b, 0, 0))
    out = pl.pallas_call(
        _loss_kernel,
        grid=(B, NBLK),
        in_specs=[
            row_spec, row_spec, pts_spec,
            row_spec, row_spec, col_spec,
            pl.BlockSpec((1, 1, 1), lambda b, i: (b, 0, 0)),
        ],
        out_specs=pl.BlockSpec((1, 1), lambda b, i: (0, 0)),
        out_shape=jax.ShapeDtypeStruct((1, 1), jnp.float32),
        scratch_shapes=[pltpu.VMEM((1, N), jnp.float32)],
    )(gt_t, pred_t, pt, gtb_t, predb_t, predb, rinv)
    return out[0, 0]


# speedup vs baseline: 1.9338x; 1.9338x over previous
import jax
import jax.numpy as jnp
from jax import lax
from jax.experimental import pallas as pl
from jax.experimental.pallas import tpu as pltpu

ALPHA_C = 0.1
K_NN = 4
RADIUS_C = 0.07
H2 = 0.03 ** 2
EPS_C = 1e-12

B, C, N = 16, 3, 2048
C8 = 8
ROWS = 256
NBLK = N // ROWS


def _loss_kernel(gt_row_ref, pred_row_ref, pt_ref,
                 gtb_row_ref, predb_row_ref, predb_ref,
                 rinv_ref, out_ref, colmin_ref):
    b = pl.program_id(0)
    i = pl.program_id(1)

    @pl.when((b == 0) & (i == 0))
    def _init_out():
        out_ref[...] = jnp.zeros((1, 1), jnp.float32)

    @pl.when(i == 0)
    def _init_colmin():
        colmin_ref[...] = jnp.full((1, N), jnp.inf, jnp.float32)

    g = gt_row_ref[0]
    q = pred_row_ref[0]
    pt = pt_ref[0]
    gb = gtb_row_ref[0]
    qb = predb_row_ref[0]
    pb = predb_ref[0]

    pt_hi = pt.astype(jnp.bfloat16)
    pt_lo = (pt - pt_hi.astype(jnp.float32)).astype(jnp.bfloat16)

    g2 = 0.25 * jnp.sum(g * g, axis=1, keepdims=True)
    q2 = 0.25 * jnp.sum(q * q, axis=1, keepdims=True)
    p2 = jnp.sum(pt * pt, axis=1)[None, :]

    ab = jnp.dot(jnp.concatenate([gb, qb], axis=0), pb,
                 preferred_element_type=jnp.float32)
    d_n = (g2 + p2) + ab[:ROWS]
    dpp_n = (q2 + p2) + ab[ROWS:]

    rinv = rinv_ref[0, 0, 0]
    inv_bn = 1.0 / (B * N)

    rowmin = jnp.min(d_n, axis=1)
    colmin_ref[...] = jnp.minimum(colmin_ref[...],
                                  jnp.min(d_n, axis=0, keepdims=True))
    acc = (0.8 * inv_bn) * rinv * jnp.sum(rowmin)

    qa = -0.5 * q
    m = jnp.min(dpp_n, axis=1, keepdims=True)
    dpp_n = jnp.where(dpp_n == m, jnp.inf, dpp_n)
    rep = jnp.zeros((), jnp.float32)
    for _ in range(K_NN):
        m = jnp.min(dpp_n, axis=1, keepdims=True)
        sel = dpp_n == m
        selb = sel.astype(jnp.bfloat16)
        c = (jnp.dot(selb, pt_hi, preferred_element_type=jnp.float32)
             + jnp.dot(selb, pt_lo, preferred_element_type=jnp.float32))
        dpp_n = jnp.where(sel, jnp.inf, dpp_n)
        diff = c - qa
        d2 = jnp.maximum(jnp.sum(diff * diff, axis=1), EPS_C)
        dist = jnp.sqrt(d2)
        w = jnp.exp(-d2 * (1.0 / H2))
        rep = rep + jnp.sum((RADIUS_C - dist) * w)
    acc = acc + (ALPHA_C * inv_bn / K_NN) * rep

    tail = jnp.where(i == NBLK - 1,
                     (0.2 * inv_bn) * rinv * jnp.sum(colmin_ref[...]),
                     0.0)
    out_ref[...] = out_ref[...] + (acc + tail)


def kernel(pred, gt, pcd_radius):
    pad_t = [(0, 0), (0, 0), (0, C8 - C)]
    gt_t = jnp.pad(jnp.transpose(-2.0 * gt, (0, 2, 1)), pad_t)
    pred_t = jnp.pad(jnp.transpose(-2.0 * pred, (0, 2, 1)), pad_t)
    pt = jnp.pad(jnp.transpose(pred, (0, 2, 1)), pad_t)
    gtb_t = gt_t.astype(jnp.bfloat16)
    predb_t = pred_t.astype(jnp.bfloat16)
    predb = jnp.pad(pred, [(0, 0), (0, C8 - C), (0, 0)]).astype(jnp.bfloat16)
    rinv = (1.0 / pcd_radius).reshape(B, 1, 1)
    row_spec = pl.BlockSpec((1, ROWS, C8), lambda b, i: (b, i, 0))
    pts_spec = pl.BlockSpec((1, N, C8), lambda b, i: (b, 0, 0))
    col_spec = pl.BlockSpec((1, C8, N), lambda b, i: (b, 0, 0))
    out = pl.pallas_call(
        _loss_kernel,
        grid=(B, NBLK),
        in_specs=[
            row_spec, row_spec, pts_spec,
            row_spec, row_spec, col_spec,
            pl.BlockSpec((1, 1, 1), lambda b, i: (b, 0, 0)),
        ],
        out_specs=pl.BlockSpec((1, 1), lambda b, i: (0, 0)),
        out_shape=jax.ShapeDtypeStruct((1, 1), jnp.float32),
        scratch_shapes=[pltpu.VMEM((1, N), jnp.float32)],
    )(gt_t, pred_t, pt, gtb_t, predb_t, predb, rinv)
    return out[0, 0]


# speedup vs baseline: 2.5575x; 1.3225x over previous
import jax
import jax.numpy as jnp
from jax import lax
from jax.experimental import pallas as pl
from jax.experimental.pallas import tpu as pltpu

ALPHA_C = 0.1
K_NN = 4
RADIUS_C = 0.07
H2 = 0.03 ** 2
EPS_C = 1e-12

B, C, N = 16, 3, 2048
C8 = 8
K32 = 4 * C8
ROWS = 256
NBLK = N // ROWS


def _loss_kernel(gt_row_ref, pred_row_ref, pred_ref,
                 lhs_g_ref, lhs_qn_ref, lhs_qe_ref, rhs_ref,
                 rinv_ref, out_ref, colmin_ref):
    b = pl.program_id(0)
    i = pl.program_id(1)

    @pl.when((b == 0) & (i == 0))
    def _init_out():
        out_ref[...] = jnp.zeros((1, 1), jnp.float32)

    @pl.when(i == 0)
    def _init_colmin():
        colmin_ref[...] = jnp.full((1, N), jnp.inf, jnp.float32)

    g = gt_row_ref[0]
    q = pred_row_ref[0]
    p = pred_ref[0]

    g2 = 0.25 * jnp.sum(g * g, axis=1, keepdims=True)
    q2 = 0.25 * jnp.sum(q * q, axis=1, keepdims=True)
    p2 = jnp.sum(p * p, axis=0, keepdims=True)

    lhs = jnp.concatenate([lhs_g_ref[0], lhs_qn_ref[0], lhs_qe_ref[0]],
                          axis=0)
    ab = jnp.dot(lhs, rhs_ref[0],
                 preferred_element_type=jnp.float32)
    qp2 = q2 + p2
    d_n = (g2 + p2) + ab[:ROWS]
    dpp_n = qp2 + ab[ROWS:2 * ROWS]
    dpp_e = qp2 + ab[2 * ROWS:]

    rinv = rinv_ref[0, 0, 0]
    inv_bn = 1.0 / (B * N)

    rowmin = jnp.min(d_n, axis=1)
    colmin_ref[...] = jnp.minimum(colmin_ref[...],
                                  jnp.min(d_n, axis=0, keepdims=True))
    acc = (0.8 * inv_bn) * rinv * jnp.sum(rowmin)

    m = jnp.min(dpp_n, axis=1, keepdims=True)
    dpp_n = jnp.where(dpp_n == m, jnp.inf, dpp_n)
    rep = jnp.zeros((), jnp.float32)
    for _ in range(K_NN):
        m = jnp.min(dpp_n, axis=1, keepdims=True)
        sel = dpp_n == m
        e = jnp.min(jnp.where(sel, dpp_e, jnp.inf), axis=1, keepdims=True)
        dpp_n = jnp.where(sel, jnp.inf, dpp_n)
        d2 = jnp.maximum(e, EPS_C)
        dist = jnp.sqrt(d2)
        w = jnp.exp(-d2 * (1.0 / H2))
        rep = rep + jnp.sum((RADIUS_C - dist) * w)
    acc = acc + (ALPHA_C * inv_bn / K_NN) * rep

    tail = jnp.where(i == NBLK - 1,
                     (0.2 * inv_bn) * rinv * jnp.sum(colmin_ref[...]),
                     0.0)
    out_ref[...] = out_ref[...] + (acc + tail)


def kernel(pred, gt, pcd_radius):
    f32 = jnp.float32
    bf16 = jnp.bfloat16
    pad_t = [(0, 0), (0, 0), (0, C8 - C)]
    pad_c = [(0, 0), (0, C8 - C), (0, 0)]
    gt_t = jnp.pad(jnp.transpose(-2.0 * gt, (0, 2, 1)), pad_t)
    pred_t = jnp.pad(jnp.transpose(-2.0 * pred, (0, 2, 1)), pad_t)
    pred_p = jnp.pad(pred, pad_c)

    zeros24 = jnp.zeros((B, N, K32 - C8), bf16)
    gb = gt_t.astype(bf16)
    qh = pred_t.astype(bf16)
    ql = (pred_t - qh.astype(f32)).astype(bf16)
    lhs_g = jnp.concatenate([gb, zeros24], axis=2)
    lhs_qn = jnp.concatenate([qh, zeros24], axis=2)
    lhs_qe = jnp.concatenate([qh, ql, qh, ql], axis=2)
    ph = pred_p.astype(bf16)
    pl_ = (pred_p - ph.astype(f32)).astype(bf16)
    rhs = jnp.concatenate([ph, pl_, pl_, ph], axis=1)

    rinv = (1.0 / pcd_radius).reshape(B, 1, 1)
    row_spec = pl.BlockSpec((1, ROWS, C8), lambda b, i: (b, i, 0))
    lhs_spec = pl.BlockSpec((1, ROWS, K32), lambda b, i: (b, i, 0))
    col_spec = pl.BlockSpec((1, C8, N), lambda b, i: (b, 0, 0))
    rhs_spec = pl.BlockSpec((1, K32, N), lambda b, i: (b, 0, 0))
    out = pl.pallas_call(
        _loss_kernel,
        grid=(B, NBLK),
        in_specs=[
            row_spec, row_spec, col_spec,
            lhs_spec, lhs_spec, lhs_spec, rhs_spec,
            pl.BlockSpec((1, 1, 1), lambda b, i: (b, 0, 0)),
        ],
        out_specs=pl.BlockSpec((1, 1), lambda b, i: (0, 0)),
        out_shape=jax.ShapeDtypeStruct((1, 1), jnp.float32),
        scratch_shapes=[pltpu.VMEM((1, N), jnp.float32)],
    )(gt_t, pred_t, pred_p, lhs_g, lhs_qn, lhs_qe, rhs, rinv)
    return out[0, 0]


# speedup vs baseline: 3.4105x; 1.3335x over previous
import jax
import jax.numpy as jnp
from jax import lax
from jax.experimental import pallas as pl
from jax.experimental.pallas import tpu as pltpu

ALPHA_C = 0.1
K_NN = 4
RADIUS_C = 0.07
H2 = 0.03 ** 2
EPS_C = 1e-12

B, C, N = 16, 3, 2048
C8 = 8
K32 = 4 * C8
ROWS = 256
NBLK = N // ROWS


def _loss_kernel(gt_row_ref, pred_row_ref, pred_ref,
                 lhs_g_ref, lhs_qn_ref, lhs_qe_ref, rhs_ref,
                 rinv_ref, out_ref, colmin_ref):
    b = pl.program_id(0)
    i = pl.program_id(1)

    @pl.when((b == 0) & (i == 0))
    def _init_out():
        out_ref[...] = jnp.zeros((1, 1), jnp.float32)

    @pl.when(i == 0)
    def _init_colmin():
        colmin_ref[...] = jnp.full((1, N), jnp.inf, jnp.float32)

    g = gt_row_ref[0]
    q = pred_row_ref[0]
    p = pred_ref[0]

    g2 = 0.25 * jnp.sum(g * g, axis=1, keepdims=True)
    q2 = 0.25 * jnp.sum(q * q, axis=1, keepdims=True)
    p2 = jnp.sum(p * p, axis=0, keepdims=True)

    lhs = jnp.concatenate([lhs_g_ref[0], lhs_qn_ref[0], lhs_qe_ref[0]],
                          axis=0)
    ab = jnp.dot(lhs, rhs_ref[0],
                 preferred_element_type=jnp.float32)
    qp2 = q2 + p2
    d_n = (g2 + p2) + ab[:ROWS]
    dpp_n = qp2 + ab[ROWS:2 * ROWS]

    dpp_nb = dpp_n.astype(jnp.bfloat16)
    delta = ((qp2 + ab[2 * ROWS:]) - dpp_nb.astype(jnp.float32)
             ).astype(jnp.bfloat16)

    rinv = rinv_ref[0, 0, 0]
    inv_bn = 1.0 / (B * N)

    rowmin = jnp.min(d_n, axis=1)
    colmin_ref[...] = jnp.minimum(colmin_ref[...],
                                  jnp.min(d_n, axis=0, keepdims=True))
    acc = (0.8 * inv_bn) * rinv * jnp.sum(rowmin)

    inf_b = jnp.array(jnp.inf, jnp.bfloat16)
    m = jnp.min(dpp_nb, axis=1, keepdims=True)
    dpp_nb = jnp.where(dpp_nb == m, inf_b, dpp_nb)
    rep = jnp.zeros((), jnp.float32)
    for _ in range(K_NN):
        m = jnp.min(dpp_nb, axis=1, keepdims=True)
        sel = dpp_nb == m
        db = jnp.min(jnp.where(sel, delta, inf_b), axis=1, keepdims=True)
        dpp_nb = jnp.where(sel, inf_b, dpp_nb)
        e = m.astype(jnp.float32) + db.astype(jnp.float32)
        d2 = jnp.maximum(e, EPS_C)
        dist = jnp.sqrt(d2)
        w = jnp.exp(-d2 * (1.0 / H2))
        rep = rep + jnp.sum((RADIUS_C - dist) * w)
    acc = acc + (ALPHA_C * inv_bn / K_NN) * rep

    tail = jnp.where(i == NBLK - 1,
                     (0.2 * inv_bn) * rinv * jnp.sum(colmin_ref[...]),
                     0.0)
    out_ref[...] = out_ref[...] + (acc + tail)


def kernel(pred, gt, pcd_radius):
    f32 = jnp.float32
    bf16 = jnp.bfloat16
    pad_t = [(0, 0), (0, 0), (0, C8 - C)]
    pad_c = [(0, 0), (0, C8 - C), (0, 0)]
    gt_t = jnp.pad(jnp.transpose(-2.0 * gt, (0, 2, 1)), pad_t)
    pred_t = jnp.pad(jnp.transpose(-2.0 * pred, (0, 2, 1)), pad_t)
    pred_p = jnp.pad(pred, pad_c)

    zeros24 = jnp.zeros((B, N, K32 - C8), bf16)
    gb = gt_t.astype(bf16)
    qh = pred_t.astype(bf16)
    ql = (pred_t - qh.astype(f32)).astype(bf16)
    lhs_g = jnp.concatenate([gb, zeros24], axis=2)
    lhs_qn = jnp.concatenate([qh, zeros24], axis=2)
    lhs_qe = jnp.concatenate([qh, ql, qh, ql], axis=2)
    ph = pred_p.astype(bf16)
    pl_ = (pred_p - ph.astype(f32)).astype(bf16)
    rhs = jnp.concatenate([ph, pl_, pl_, ph], axis=1)

    rinv = (1.0 / pcd_radius).reshape(B, 1, 1)
    row_spec = pl.BlockSpec((1, ROWS, C8), lambda b, i: (b, i, 0))
    lhs_spec = pl.BlockSpec((1, ROWS, K32), lambda b, i: (b, i, 0))
    col_spec = pl.BlockSpec((1, C8, N), lambda b, i: (b, 0, 0))
    rhs_spec = pl.BlockSpec((1, K32, N), lambda b, i: (b, 0, 0))
    out = pl.pallas_call(
        _loss_kernel,
        grid=(B, NBLK),
        in_specs=[
            row_spec, row_spec, col_spec,
            lhs_spec, lhs_spec, lhs_spec, rhs_spec,
            pl.BlockSpec((1, 1, 1), lambda b, i: (b, 0, 0)),
        ],
        out_specs=pl.BlockSpec((1, 1), lambda b, i: (0, 0)),
        out_shape=jax.ShapeDtypeStruct((1, 1), jnp.float32),
        scratch_shapes=[pltpu.VMEM((1, N), jnp.float32)],
    )(gt_t, pred_t, pred_p, lhs_g, lhs_qn, lhs_qe, rhs, rinv)
    return out[0, 0]
